# baseline (device time: 17238 ns/iter reference)
import jax
import jax.numpy as jnp
from jax import lax
from jax.experimental import pallas as pl
from jax.experimental.pallas import tpu as pltpu

N_DEV = 32


def kernel(x, w_mat):
    m_per, k = x.shape
    n = w_mat.shape[1]
    n_per = n // N_DEV

    def body(x_hbm, w_hbm, out_ref, xv, wv, y_ref, recv_ref,
             copy_sems, send_sems, recv_sems, cred_sems):
        my = lax.axis_index("i")

        barrier = pltpu.get_barrier_semaphore()
        pl.semaphore_signal(barrier, inc=1)
        pl.semaphore_wait(barrier, 1)

        for d in range(1, N_DEV):
            s = lax.rem(my - d + N_DEV, N_DEV)
            pl.semaphore_signal(
                cred_sems.at[d - 1], inc=1, device_id=s,
                device_id_type=pl.DeviceIdType.LOGICAL,
            )

        cx = pltpu.make_async_copy(x_hbm, xv, copy_sems.at[0])
        cw = pltpu.make_async_copy(w_hbm, wv, copy_sems.at[1])
        cx.start()
        cw.start()
        cx.wait()
        cw.wait()

        y = jnp.dot(xv[...], wv[...], preferred_element_type=jnp.float32)
        c = 0.7978845608028654
        y = 0.5 * y * (1.0 + jnp.tanh(c * (y + 0.044715 * y * y * y)))
        y = y.astype(jnp.bfloat16)
        for t in range(N_DEV):
            y_ref[t] = y[:, t * n_per:(t + 1) * n_per]

        sends = []
        for d in range(1, N_DEV):
            t = lax.rem(my + d, N_DEV)
            pl.semaphore_wait(cred_sems.at[d - 1], 1)
            rdma = pltpu.make_async_remote_copy(
                src_ref=y_ref.at[t],
                dst_ref=recv_ref.at[my],
                send_sem=send_sems.at[d - 1],
                recv_sem=recv_sems.at[d - 1],
                device_id=t,
                device_id_type=pl.DeviceIdType.LOGICAL,
            )
            sends.append(rdma)
            rdma.start()

        recv_ref[my] = y_ref[my]

        for d in range(1, N_DEV):
            s = lax.rem(my - d + N_DEV, N_DEV)
            recv = pltpu.make_async_remote_copy(
                src_ref=y_ref.at[s],
                dst_ref=recv_ref.at[s],
                send_sem=send_sems.at[d - 1],
                recv_sem=recv_sems.at[d - 1],
                device_id=s,
                device_id_type=pl.DeviceIdType.LOGICAL,
            )
            recv.wait_recv()

        out_ref[...] = recv_ref[...].reshape(N_DEV * m_per, n_per).astype(
            jnp.float32
        )

        for rdma in sends:
            rdma.wait_send()

    return pl.pallas_call(
        body,
        out_shape=jax.ShapeDtypeStruct((N_DEV * m_per, n_per), jnp.float32),
        in_specs=[
            pl.BlockSpec(memory_space=pltpu.MemorySpace.HBM),
            pl.BlockSpec(memory_space=pltpu.MemorySpace.HBM),
        ],
        out_specs=pl.BlockSpec(memory_space=pltpu.VMEM),
        scratch_shapes=[
            pltpu.VMEM((m_per, k), x.dtype),
            pltpu.VMEM((k, n), w_mat.dtype),
            pltpu.VMEM((N_DEV, m_per, n_per), jnp.bfloat16),
            pltpu.VMEM((N_DEV, m_per, n_per), jnp.bfloat16),
            pltpu.SemaphoreType.DMA((2,)),
            pltpu.SemaphoreType.DMA((N_DEV - 1,)),
            pltpu.SemaphoreType.DMA((N_DEV - 1,)),
            pltpu.SemaphoreType.REGULAR((N_DEV - 1,)),
        ],
        compiler_params=pltpu.CompilerParams(collective_id=0),
    )(x, w_mat)


# device time: 17220 ns/iter; 1.0010x vs baseline; 1.0010x over previous
import jax
import jax.numpy as jnp
from jax import lax
from jax.experimental import pallas as pl
from jax.experimental.pallas import tpu as pltpu

N_DEV = 32


def kernel(x, w_mat):
    m_per, k = x.shape
    n = w_mat.shape[1]
    n_per = n // N_DEV

    def body(x_ref, w_ref, out_ref, y_ref, recv_ref,
             send_sems, recv_sems, cred_sems):
        my = lax.axis_index("i")

        barrier = pltpu.get_barrier_semaphore()
        pl.semaphore_signal(barrier, inc=1)
        pl.semaphore_wait(barrier, 1)

        for d in range(1, N_DEV):
            s = lax.rem(my - d + N_DEV, N_DEV)
            pl.semaphore_signal(
                cred_sems.at[d - 1], inc=1, device_id=s,
                device_id_type=pl.DeviceIdType.LOGICAL,
            )

        y = jnp.dot(x_ref[...], w_ref[...], preferred_element_type=jnp.float32)
        c = 0.7978845608028654
        y = 0.5 * y * (1.0 + jnp.tanh(c * (y + 0.044715 * y * y * y)))
        y = y.astype(jnp.bfloat16)
        for t in range(N_DEV):
            y_ref[t] = y[:, t * n_per:(t + 1) * n_per]

        sends = []
        for d in range(1, N_DEV):
            t = lax.rem(my + d, N_DEV)
            pl.semaphore_wait(cred_sems.at[d - 1], 1)
            rdma = pltpu.make_async_remote_copy(
                src_ref=y_ref.at[t],
                dst_ref=recv_ref.at[my],
                send_sem=send_sems.at[d - 1],
                recv_sem=recv_sems.at[d - 1],
                device_id=t,
                device_id_type=pl.DeviceIdType.LOGICAL,
            )
            sends.append(rdma)
            rdma.start()

        recv_ref[my] = y_ref[my]

        for d in range(1, N_DEV):
            s = lax.rem(my - d + N_DEV, N_DEV)
            recv = pltpu.make_async_remote_copy(
                src_ref=y_ref.at[s],
                dst_ref=recv_ref.at[s],
                send_sem=send_sems.at[d - 1],
                recv_sem=recv_sems.at[d - 1],
                device_id=s,
                device_id_type=pl.DeviceIdType.LOGICAL,
            )
            recv.wait_recv()

        out_ref[...] = recv_ref[...].reshape(N_DEV * m_per, n_per).astype(
            jnp.float32
        )

        for rdma in sends:
            rdma.wait_send()

    return pl.pallas_call(
        body,
        out_shape=jax.ShapeDtypeStruct((N_DEV * m_per, n_per), jnp.float32),
        in_specs=[
            pl.BlockSpec(memory_space=pltpu.VMEM),
            pl.BlockSpec(memory_space=pltpu.VMEM),
        ],
        out_specs=pl.BlockSpec(memory_space=pltpu.VMEM),
        scratch_shapes=[
            pltpu.VMEM((N_DEV, m_per, n_per), jnp.bfloat16),
            pltpu.VMEM((N_DEV, m_per, n_per), jnp.bfloat16),
            pltpu.SemaphoreType.DMA((N_DEV - 1,)),
            pltpu.SemaphoreType.DMA((N_DEV - 1,)),
            pltpu.SemaphoreType.REGULAR((N_DEV - 1,)),
        ],
        compiler_params=pltpu.CompilerParams(collective_id=0),
    )(x, w_mat)


# device time: 16954 ns/iter; 1.0168x vs baseline; 1.0157x over previous
import jax
import jax.numpy as jnp
from jax import lax
from jax.experimental import pallas as pl
from jax.experimental.pallas import tpu as pltpu

N_DEV = 32


def kernel(x, w_mat):
    m_per, k = x.shape
    n = w_mat.shape[1]
    n_per = n // N_DEV

    def body(x_ref, w_ref, out_ref, y_ref, recv_ref,
             send_sems, recv_sems):
        my = lax.axis_index("i")

        barrier = pltpu.get_barrier_semaphore()
        for dev in range(N_DEV):
            @pl.when(dev != my)
            def _(dev=dev):
                pl.semaphore_signal(
                    barrier, inc=1, device_id=dev,
                    device_id_type=pl.DeviceIdType.LOGICAL,
                )

        y = jnp.dot(x_ref[...], w_ref[...], preferred_element_type=jnp.float32)
        c = 0.7978845608028654
        y = 0.5 * y * (1.0 + jnp.tanh(c * (y + 0.044715 * y * y * y)))
        y = y.astype(jnp.bfloat16)
        for t in range(N_DEV):
            y_ref[t] = y[:, t * n_per:(t + 1) * n_per]

        pl.semaphore_wait(barrier, N_DEV - 1)
        sends = []
        for d in range(1, N_DEV):
            t = lax.rem(my + d, N_DEV)
            rdma = pltpu.make_async_remote_copy(
                src_ref=y_ref.at[t],
                dst_ref=recv_ref.at[my],
                send_sem=send_sems.at[d - 1],
                recv_sem=recv_sems.at[d - 1],
                device_id=t,
                device_id_type=pl.DeviceIdType.LOGICAL,
            )
            sends.append(rdma)
            rdma.start()

        recv_ref[my] = y_ref[my]

        for d in range(1, N_DEV):
            s = lax.rem(my - d + N_DEV, N_DEV)
            recv = pltpu.make_async_remote_copy(
                src_ref=y_ref.at[s],
                dst_ref=recv_ref.at[s],
                send_sem=send_sems.at[d - 1],
                recv_sem=recv_sems.at[d - 1],
                device_id=s,
                device_id_type=pl.DeviceIdType.LOGICAL,
            )
            recv.wait_recv()

        out_ref[...] = recv_ref[...].reshape(N_DEV * m_per, n_per).astype(
            jnp.float32
        )

        for rdma in sends:
            rdma.wait_send()

    return pl.pallas_call(
        body,
        out_shape=jax.ShapeDtypeStruct((N_DEV * m_per, n_per), jnp.float32),
        in_specs=[
            pl.BlockSpec(memory_space=pltpu.VMEM),
            pl.BlockSpec(memory_space=pltpu.VMEM),
        ],
        out_specs=pl.BlockSpec(memory_space=pltpu.VMEM),
        scratch_shapes=[
            pltpu.VMEM((N_DEV, m_per, n_per), jnp.bfloat16),
            pltpu.VMEM((N_DEV, m_per, n_per), jnp.bfloat16),
            pltpu.SemaphoreType.DMA((N_DEV - 1,)),
            pltpu.SemaphoreType.DMA((N_DEV - 1,)),
        ],
        compiler_params=pltpu.CompilerParams(collective_id=0),
    )(x, w_mat)
